# baseline (speedup 1.0000x reference)
import functools

import jax
import jax.numpy as jnp
from jax import lax
from jax.experimental import pallas as pl
from jax.experimental.pallas import tpu as pltpu
from jax.experimental.pallas import tpu_sc as plsc


@functools.lru_cache(maxsize=None)
def _make_gather(V, D, B):
  info = plsc.get_sparse_core_info()
  NC, NS, L = info.num_cores, info.num_subcores, info.num_lanes
  NW = NC * NS
  assert B % (2 * NW) == 0 and D % L == 0 and V % 2 == 0
  b_per_w = B // NW
  t_per_w = b_per_w // 2
  DW = D
  CH = min(64, b_per_w)
  n_ch = b_per_w // CH
  n_lc = DW // L
  mesh = plsc.VectorSubcoreMesh(core_axis_name="c", subcore_axis_name="s")

  @functools.partial(
      pl.kernel,
      out_type=jax.ShapeDtypeStruct((B, D), jnp.bfloat16),
      mesh=mesh,
      scratch_types=[
          pltpu.VMEM((b_per_w + L,), jnp.int32),
          pltpu.VMEM((b_per_w,), jnp.int32),
          pltpu.VMEM((b_per_w, DW), jnp.int32),
          pltpu.VMEM((t_per_w, DW), jnp.int32),
          pltpu.SemaphoreType.DMA,
          pltpu.SemaphoreType.DMA,
      ],
      compiler_params=pltpu.CompilerParams(needs_layout_passes=False),
  )
  def gather(table_hbm, idx_hbm, out_hbm, idx_v, widx_v, g_v, out_v, gsem,
             osem):
    wid = lax.axis_index("s") * NC + lax.axis_index("c")
    base = wid * b_per_w
    table_w = table_hbm.bitcast(jnp.int32)
    out_w = out_hbm.bitcast(jnp.int32)

    pltpu.sync_copy(idx_hbm.at[pl.ds(base, b_per_w)],
                    idx_v.at[pl.ds(0, b_per_w)])
    gcopies = []
    for c in range(n_ch):
      for i in range(CH // L):
        widx_v[pl.ds(c * CH + i * L, L)] = lax.shift_right_logical(
            idx_v[pl.ds(c * CH + i * L, L)], 1
        )
      gcopies.append(
          pltpu.async_copy(
              table_w.at[widx_v.at[pl.ds(c * CH, CH)]],
              g_v.at[pl.ds(c * CH, CH)],
              gsem,
          )
      )

    t_ch = CH // 2
    ocopies = []
    for c in range(n_ch):
      gcopies[c].wait()

      @plsc.parallel_loop(c * t_ch, (c + 1) * t_ch, 1, unroll=4)
      def fixup(t):
        rv = idx_v[pl.ds(2 * t, L)]
        sa = lax.shift_left(lax.bitwise_and(rv[0], 1), 4)
        sb = lax.shift_left(lax.bitwise_and(rv[1], 1), 4)
        for lc in range(n_lc):
          wa = g_v[2 * t, pl.ds(lc * L, L)]
          wb = g_v[2 * t + 1, pl.ds(lc * L, L)]
          pa = lax.shift_right_logical(wa, sa)
          pb = lax.shift_right_logical(wb, sb)
          w = plsc.bitcast(
              plsc.pack(pa, pb, format=plsc.PackFormat.INTERLEAVED),
              jnp.int32,
          )
          out_v[t, pl.ds(lc * L, L)] = w

      ocopies.append(
          pltpu.async_copy(
              out_v.at[pl.ds(c * t_ch, t_ch)],
              out_w.at[pl.ds(wid * t_per_w + c * t_ch, t_ch)],
              osem,
          )
      )
    for oc in ocopies:
      oc.wait()

  return gather


def kernel(input_, indices, n_tpc):
  V, D = input_.shape
  (B,) = indices.shape
  return _make_gather(V, D, B)(input_, indices)


# speedup vs baseline: 1.0946x; 1.0946x over previous
import functools

import jax
import jax.numpy as jnp
from jax import lax
from jax.experimental import pallas as pl
from jax.experimental.pallas import tpu as pltpu
from jax.experimental.pallas import tpu_sc as plsc


@functools.lru_cache(maxsize=None)
def _make_gather(V, D, B):
  info = plsc.get_sparse_core_info()
  NC, NS, L = info.num_cores, info.num_subcores, info.num_lanes
  NW = NC * NS
  assert B % (2 * NW) == 0 and D % L == 0 and V % 2 == 0
  b_per_w = B // NW
  t_per_w = b_per_w // 2
  DW = D
  CH = min(128, b_per_w)
  n_ch = b_per_w // CH
  n_lc = DW // L
  mesh = plsc.VectorSubcoreMesh(core_axis_name="c", subcore_axis_name="s")

  @functools.partial(
      pl.kernel,
      out_type=jax.ShapeDtypeStruct((B, D), jnp.bfloat16),
      mesh=mesh,
      scratch_types=[
          pltpu.VMEM((b_per_w + L,), jnp.int32),
          pltpu.VMEM((b_per_w,), jnp.int32),
          pltpu.VMEM((b_per_w, DW), jnp.int32),
          pltpu.VMEM((t_per_w, DW), jnp.int32),
          pltpu.SemaphoreType.DMA,
          pltpu.SemaphoreType.DMA,
      ],
      compiler_params=pltpu.CompilerParams(needs_layout_passes=False),
  )
  def gather(table_hbm, idx_hbm, out_hbm, idx_v, widx_v, g_v, out_v, gsem,
             osem):
    wid = lax.axis_index("s") * NC + lax.axis_index("c")
    base = wid * b_per_w
    table_w = table_hbm.bitcast(jnp.int32)
    out_w = out_hbm.bitcast(jnp.int32)

    pltpu.sync_copy(idx_hbm.at[pl.ds(base, b_per_w)],
                    idx_v.at[pl.ds(0, b_per_w)])
    gcopies = []
    for c in range(n_ch):
      for i in range(CH // L):
        widx_v[pl.ds(c * CH + i * L, L)] = lax.shift_right_logical(
            idx_v[pl.ds(c * CH + i * L, L)], 1
        )
      gcopies.append(
          pltpu.async_copy(
              table_w.at[widx_v.at[pl.ds(c * CH, CH)]],
              g_v.at[pl.ds(c * CH, CH)],
              gsem,
          )
      )

    t_ch = CH // 2
    ocopies = []
    for c in range(n_ch):
      gcopies[c].wait()

      @plsc.parallel_loop(c * t_ch, (c + 1) * t_ch, 1, unroll=2)
      def fixup(t):
        rv = idx_v[pl.ds(2 * t, L)]
        sa = lax.shift_left(lax.bitwise_and(rv[0], 1), 4)
        sb = lax.shift_left(lax.bitwise_and(rv[1], 1), 4)
        for lc in range(n_lc):
          wa = g_v[2 * t, pl.ds(lc * L, L)]
          wb = g_v[2 * t + 1, pl.ds(lc * L, L)]
          pa = lax.shift_right_logical(wa, sa)
          pb = lax.shift_right_logical(wb, sb)
          w = plsc.bitcast(
              plsc.pack(pa, pb, format=plsc.PackFormat.INTERLEAVED),
              jnp.int32,
          )
          out_v[t, pl.ds(lc * L, L)] = w

      ocopies.append(
          pltpu.async_copy(
              out_v.at[pl.ds(c * t_ch, t_ch)],
              out_w.at[pl.ds(wid * t_per_w + c * t_ch, t_ch)],
              osem,
          )
      )
    for oc in ocopies:
      oc.wait()

  return gather


def kernel(input_, indices, n_tpc):
  V, D = input_.shape
  (B,) = indices.shape
  return _make_gather(V, D, B)(input_, indices)
